# Initial kernel scaffold; baseline (speedup 1.0000x reference)
import jax
import jax.numpy as jnp
from jax import lax
from jax.experimental import pallas as pl
from jax.experimental.pallas import tpu as pltpu
from jax.experimental.pallas import tpu_sc as plsc

N = 10000
D = 128
E = 320000
NC = 2
NS = 16
NW = NC * NS
E_T = E // NW
K = 125
NCH = E_T // K
ROWS_T = N // NS
DEG_W = 16
LANES = 16


def _sc_aggregate(with_deg):
    mesh = plsc.VectorSubcoreMesh(core_axis_name="c", subcore_axis_name="s")
    out_type = [jax.ShapeDtypeStruct((NC, N, D), jnp.float32)]
    scratch = [
        pltpu.VMEM((NCH, K), jnp.int32),
        pltpu.VMEM((NCH, K), jnp.int32),
        pltpu.VMEM((K, D), jnp.float32),
        pltpu.VMEM((K, D), jnp.float32),
        pltpu.VMEM_SHARED((N, D), jnp.float32),
        pltpu.SemaphoreType.DMA,
        pltpu.SemaphoreType.DMA,
    ]
    if with_deg:
        out_type.append(jax.ShapeDtypeStruct((NC, N, DEG_W), jnp.float32))
        scratch.append(pltpu.VMEM((K, DEG_W), jnp.float32))
        scratch.append(pltpu.VMEM_SHARED((N, DEG_W), jnp.float32))

    def body(h_hbm, src_hbm, dst_hbm, agg_out, *rest):
        if with_deg:
            (deg_out, src_v, dst_v, buf0, buf1, acc_sh, sem0, sem1,
             ones_v, deg_sh) = rest
        else:
            src_v, dst_v, buf0, buf1, acc_sh, sem0, sem1 = rest
        c = lax.axis_index("c")
        s = lax.axis_index("s")
        wid = s * NC + c

        pltpu.sync_copy(src_hbm.at[wid], src_v)
        pltpu.sync_copy(dst_hbm.at[wid], dst_v)

        zero16 = jnp.zeros((LANES,), jnp.float32)

        def zrow(r, carry):
            for cc in range(D // LANES):
                buf0[r, pl.ds(cc * LANES, LANES)] = zero16
            return carry

        lax.fori_loop(0, K, zrow, 0)
        base = s * ROWS_T
        for i in range(ROWS_T // K):
            pltpu.sync_copy(buf0, acc_sh.at[pl.ds(base + i * K, K)])
        if with_deg:
            def zdrow(r, carry):
                ones_v[r, pl.ds(0, LANES)] = zero16
                return carry

            lax.fori_loop(0, K, zdrow, 0)
            for i in range(ROWS_T // K):
                pltpu.sync_copy(ones_v, deg_sh.at[pl.ds(base + i * K, K)])
            one16 = jnp.ones((LANES,), jnp.float32)

            def orow(r, carry):
                ones_v[r, pl.ds(0, LANES)] = one16
                return carry

            lax.fori_loop(0, K, orow, 0)
        plsc.subcore_barrier()

        bufs = (buf0, buf1)
        sems = (sem0, sem1)
        pltpu.async_copy(h_hbm.at[src_v.at[0]], buf0, sem0)
        pltpu.async_copy(h_hbm.at[src_v.at[1]], buf1, sem1)

        def step(i, carry):
            for b in range(2):
                j = 2 * i + b
                pltpu.make_async_copy(h_hbm.at[src_v.at[j]], bufs[b],
                                      sems[b]).wait()
                pltpu.sync_copy(bufs[b], acc_sh.at[dst_v.at[j]], add=True)
                if with_deg:
                    pltpu.sync_copy(ones_v, deg_sh.at[dst_v.at[j]], add=True)
                jn = lax.rem(j + 2, NCH)
                pltpu.async_copy(h_hbm.at[src_v.at[jn]], bufs[b], sems[b])
            return carry

        lax.fori_loop(0, NCH // 2, step, 0)
        pltpu.make_async_copy(h_hbm.at[src_v.at[0]], buf0, sem0).wait()
        pltpu.make_async_copy(h_hbm.at[src_v.at[1]], buf1, sem1).wait()
        plsc.subcore_barrier()

        pltpu.sync_copy(acc_sh.at[pl.ds(base, ROWS_T)],
                        agg_out.at[c, pl.ds(base, ROWS_T)])
        if with_deg:
            pltpu.sync_copy(deg_sh.at[pl.ds(base, ROWS_T)],
                            deg_out.at[c, pl.ds(base, ROWS_T)])

    return pl.kernel(body, out_type=out_type, mesh=mesh,
                     scratch_types=scratch)


_sc_agg_deg = _sc_aggregate(with_deg=True)
_sc_agg = _sc_aggregate(with_deg=False)


def _tc_sage(h_self, agg_parts, deg_parts, W_self, W_neigh, b, relu):
    R = 1000
    grid = N // R

    def body(x_ref, agg_ref, deg_ref, ws_ref, wn_ref, b_ref, o_ref):
        agg = agg_ref[0] + agg_ref[1]
        deg = jnp.maximum(deg_ref[0, :, :1] + deg_ref[1, :, :1], 1.0)
        hn = agg / deg
        out = (jnp.dot(x_ref[...], ws_ref[...],
                       preferred_element_type=jnp.float32)
               + jnp.dot(hn, wn_ref[...], preferred_element_type=jnp.float32)
               + b_ref[...])
        if relu:
            out = jnp.maximum(out, 0.0)
        o_ref[...] = out

    return pl.pallas_call(
        body,
        grid=(grid,),
        in_specs=[
            pl.BlockSpec((R, D), lambda i: (i, 0)),
            pl.BlockSpec((NC, R, D), lambda i: (0, i, 0)),
            pl.BlockSpec((NC, R, DEG_W), lambda i: (0, i, 0)),
            pl.BlockSpec((D, D), lambda i: (0, 0)),
            pl.BlockSpec((D, D), lambda i: (0, 0)),
            pl.BlockSpec((1, D), lambda i: (0, 0)),
        ],
        out_specs=pl.BlockSpec((R, D), lambda i: (i, 0)),
        out_shape=jax.ShapeDtypeStruct((N, D), jnp.float32),
    )(h_self, agg_parts, deg_parts, W_self, W_neigh, b.reshape(1, D))


def kernel(inputs, edge_index, W_self1, W_neigh1, b1, W_self2, W_neigh2, b2):
    ei = edge_index.astype(jnp.int32)
    src3 = ei[0].reshape(NW, NCH, K)
    dst3 = ei[1].reshape(NW, NCH, K)
    agg1, deg = _sc_agg_deg(inputs, src3, dst3)
    h1 = _tc_sage(inputs, agg1, deg, W_self1, W_neigh1, b1, relu=True)
    agg2 = _sc_agg(h1, src3, dst3)
    h2 = _tc_sage(h1, agg2, deg, W_self2, W_neigh2, b2, relu=False)
    return h2


# speedup vs baseline: 8.7491x; 8.7491x over previous
import jax
import jax.numpy as jnp
from jax import lax
from jax.experimental import pallas as pl
from jax.experimental.pallas import tpu as pltpu
from jax.experimental.pallas import tpu_sc as plsc

N = 10000
D = 128
E = 320000
NC = 2
NS = 16
NW = NC * NS
E_T = E // NW
K = 125
NCH = E_T // K
ROWS_T = N // NS
LANES = 16
NPAD = 10240
EPAD = 10240
TRASH = N
DSL = NPAD // NS

_MESH = plsc.VectorSubcoreMesh(core_axis_name="c", subcore_axis_name="s")


def _make_sc_agg():
    scratch = [
        pltpu.VMEM((NCH, K), jnp.int32),
        pltpu.VMEM((NCH, K), jnp.int32),
        pltpu.VMEM((K, D), jnp.float32),
        pltpu.VMEM_SHARED((N, D), jnp.float32),
        pltpu.SemaphoreType.DMA,
    ]

    def body(h_hbm, src_hbm, dst_hbm, agg_out,
             src_v, dst_v, buf0, acc_sh, sem0):
        c = lax.axis_index("c")
        s = lax.axis_index("s")
        wid = s * NC + c

        pltpu.sync_copy(src_hbm.at[wid], src_v)
        pltpu.sync_copy(dst_hbm.at[wid], dst_v)

        zero16 = jnp.zeros((LANES,), jnp.float32)

        def zrow(r, carry):
            for cc in range(D // LANES):
                buf0[r, pl.ds(cc * LANES, LANES)] = zero16
            return carry

        lax.fori_loop(0, K, zrow, 0)
        base = s * ROWS_T
        for i in range(ROWS_T // K):
            pltpu.sync_copy(buf0, acc_sh.at[pl.ds(base + i * K, K)])
        plsc.subcore_barrier()

        def step(j, carry):
            pltpu.async_copy(h_hbm.at[src_v.at[j]], buf0, sem0).wait()
            pltpu.sync_copy(buf0, acc_sh.at[dst_v.at[j]], add=True)
            return carry

        lax.fori_loop(0, NCH, step, 0)
        plsc.subcore_barrier()

        pltpu.sync_copy(acc_sh.at[pl.ds(base, ROWS_T)], agg_out.at[c, s])

    return pl.kernel(
        body,
        out_type=jax.ShapeDtypeStruct((NC, NS, ROWS_T, D), jnp.float32),
        mesh=_MESH, scratch_types=scratch)


def _make_sc_deg():
    scratch = [
        pltpu.VMEM((EPAD,), jnp.int32),
        pltpu.VMEM((NPAD,), jnp.float32),
        pltpu.VMEM((NS, DSL), jnp.float32),
        pltpu.VMEM((DSL,), jnp.float32),
        pltpu.VMEM_SHARED((NS, NPAD), jnp.float32),
    ]

    def body(dst_hbm, deg_out, dst_v, cnt_v, red_v, row_v, stage_sh):
        c = lax.axis_index("c")
        s = lax.axis_index("s")
        wid = s * NC + c

        pltpu.sync_copy(dst_hbm.at[wid], dst_v)
        zero16 = jnp.zeros((LANES,), jnp.float32)

        def zstep(i, carry):
            cnt_v[pl.ds(i * LANES, LANES)] = zero16
            return carry

        lax.fori_loop(0, NPAD // LANES, zstep, 0)
        one16 = jnp.ones((LANES,), jnp.float32)

        def cstep(i, carry):
            iv = dst_v[pl.ds(i * LANES, LANES)]
            plsc.addupdate_scatter(cnt_v, [iv], one16)
            return carry

        lax.fori_loop(0, EPAD // LANES, cstep, 0)
        pltpu.sync_copy(cnt_v, stage_sh.at[s])
        plsc.subcore_barrier()

        pltpu.sync_copy(stage_sh.at[:, pl.ds(DSL * s, DSL)], red_v)

        def rstep(k, carry):
            acc = red_v[0, pl.ds(LANES * k, LANES)]
            for t in range(1, NS):
                acc = acc + red_v[t, pl.ds(LANES * k, LANES)]
            row_v[pl.ds(LANES * k, LANES)] = acc
            return carry

        lax.fori_loop(0, DSL // LANES, rstep, 0)
        pltpu.sync_copy(row_v, deg_out.at[c, s])

    return pl.kernel(
        body,
        out_type=jax.ShapeDtypeStruct((NC, NS, DSL), jnp.float32),
        mesh=_MESH, scratch_types=scratch,
        compiler_params=pltpu.CompilerParams(needs_layout_passes=False))


_sc_agg = _make_sc_agg()
_sc_deg = _make_sc_deg()


def _tc_sage(h_self, agg_parts, deg_parts, W_self, W_neigh, b, relu):
    R = 1000
    grid = N // R

    def body(x_ref, agg_ref, deg_ref, ws_ref, wn_ref, b_ref, o_ref):
        agg = agg_ref[0] + agg_ref[1]
        deg = jnp.maximum(deg_ref[0, :, :1] + deg_ref[1, :, :1], 1.0)
        hn = agg / deg
        out = (jnp.dot(x_ref[...], ws_ref[...],
                       preferred_element_type=jnp.float32)
               + jnp.dot(hn, wn_ref[...], preferred_element_type=jnp.float32)
               + b_ref[...])
        if relu:
            out = jnp.maximum(out, 0.0)
        o_ref[...] = out

    return pl.pallas_call(
        body,
        grid=(grid,),
        in_specs=[
            pl.BlockSpec((R, D), lambda i: (i, 0)),
            pl.BlockSpec((NC, R, D), lambda i: (0, i, 0)),
            pl.BlockSpec((NC, R, 1), lambda i: (0, i, 0)),
            pl.BlockSpec((D, D), lambda i: (0, 0)),
            pl.BlockSpec((D, D), lambda i: (0, 0)),
            pl.BlockSpec((1, D), lambda i: (0, 0)),
        ],
        out_specs=pl.BlockSpec((R, D), lambda i: (i, 0)),
        out_shape=jax.ShapeDtypeStruct((N, D), jnp.float32),
    )(h_self, agg_parts, deg_parts, W_self, W_neigh, b.reshape(1, D))


def kernel(inputs, edge_index, W_self1, W_neigh1, b1, W_self2, W_neigh2, b2):
    ei = edge_index.astype(jnp.int32)
    src3 = ei[0].reshape(NW, NCH, K)
    dst3 = ei[1].reshape(NW, NCH, K)
    dst_pad = jnp.concatenate(
        [ei[1].reshape(NW, E_T),
         jnp.full((NW, EPAD - E_T), TRASH, jnp.int32)], axis=1)
    deg = _sc_deg(dst_pad).reshape(NC, NPAD)[:, :N, None]
    agg1 = _sc_agg(inputs, src3, dst3).reshape(NC, N, D)
    h1 = _tc_sage(inputs, agg1, deg, W_self1, W_neigh1, b1, relu=True)
    agg2 = _sc_agg(h1, src3, dst3).reshape(NC, N, D)
    h2 = _tc_sage(h1, agg2, deg, W_self2, W_neigh2, b2, relu=False)
    return h2


# speedup vs baseline: 8.8063x; 1.0065x over previous
import jax
import jax.numpy as jnp
from jax import lax
from jax.experimental import pallas as pl
from jax.experimental.pallas import tpu as pltpu
from jax.experimental.pallas import tpu_sc as plsc

N = 10000
D = 128
E = 320000
NC = 2
NS = 16
NW = NC * NS
E_T = E // NW
K = 125
NCH = E_T // K
ROWS_T = N // NS
LANES = 16
NPAD = 10240
DSL = NPAD // NS

_MESH = plsc.VectorSubcoreMesh(core_axis_name="c", subcore_axis_name="s")


def _make_sc_agg():
    scratch = [
        pltpu.VMEM((NCH, K), jnp.int32),
        pltpu.VMEM((NCH, K), jnp.int32),
        pltpu.VMEM((K, D), jnp.float32),
        pltpu.VMEM_SHARED((N, D), jnp.float32),
        pltpu.SemaphoreType.DMA,
    ]

    def body(h_hbm, src_hbm, dst_hbm, agg_out,
             src_v, dst_v, buf0, acc_sh, sem0):
        c = lax.axis_index("c")
        s = lax.axis_index("s")
        wid = s * NC + c

        pltpu.sync_copy(src_hbm.at[wid], src_v)
        pltpu.sync_copy(dst_hbm.at[wid], dst_v)

        zero16 = jnp.zeros((LANES,), jnp.float32)

        def zrow(r, carry):
            for cc in range(D // LANES):
                buf0[r, pl.ds(cc * LANES, LANES)] = zero16
            return carry

        lax.fori_loop(0, K, zrow, 0)
        base = s * ROWS_T
        for i in range(ROWS_T // K):
            pltpu.sync_copy(buf0, acc_sh.at[pl.ds(base + i * K, K)])
        plsc.subcore_barrier()

        def step(j, carry):
            pltpu.async_copy(h_hbm.at[src_v.at[j]], buf0, sem0).wait()
            pltpu.sync_copy(buf0, acc_sh.at[dst_v.at[j]], add=True)
            return carry

        lax.fori_loop(0, NCH, step, 0)
        plsc.subcore_barrier()

        pltpu.sync_copy(acc_sh.at[pl.ds(base, ROWS_T)], agg_out.at[c, s])

    return pl.kernel(
        body,
        out_type=jax.ShapeDtypeStruct((NC, NS, ROWS_T, D), jnp.float32),
        mesh=_MESH, scratch_types=scratch)


def _make_sc_deg():
    scratch = [
        pltpu.VMEM((E_T,), jnp.int32),
        pltpu.VMEM((NPAD,), jnp.float32),
        pltpu.VMEM((NS, DSL), jnp.float32),
        pltpu.VMEM((DSL,), jnp.float32),
        pltpu.VMEM_SHARED((NS, NPAD), jnp.float32),
    ]

    def body(dst_hbm, deg_out, dst_v, cnt_v, red_v, row_v, stage_sh):
        c = lax.axis_index("c")
        s = lax.axis_index("s")
        wid = s * NC + c

        pltpu.sync_copy(dst_hbm.at[wid], dst_v)
        zero16 = jnp.zeros((LANES,), jnp.float32)

        def zstep(i, carry):
            cnt_v[pl.ds(i * LANES, LANES)] = zero16
            return carry

        lax.fori_loop(0, NPAD // LANES, zstep, 0)
        one16 = jnp.ones((LANES,), jnp.float32)

        def cstep(i, carry):
            iv = dst_v[pl.ds(i * LANES, LANES)]
            plsc.addupdate_scatter(cnt_v, [iv], one16)
            return carry

        lax.fori_loop(0, E_T // LANES, cstep, 0)
        pltpu.sync_copy(cnt_v, stage_sh.at[s])
        plsc.subcore_barrier()

        pltpu.sync_copy(stage_sh.at[:, pl.ds(DSL * s, DSL)], red_v)

        def rstep(k, carry):
            acc = red_v[0, pl.ds(LANES * k, LANES)]
            for t in range(1, NS):
                acc = acc + red_v[t, pl.ds(LANES * k, LANES)]
            row_v[pl.ds(LANES * k, LANES)] = acc
            return carry

        lax.fori_loop(0, DSL // LANES, rstep, 0)
        pltpu.sync_copy(row_v, deg_out.at[c, s])

    return pl.kernel(
        body,
        out_type=jax.ShapeDtypeStruct((NC, NS, DSL), jnp.float32),
        mesh=_MESH, scratch_types=scratch,
        compiler_params=pltpu.CompilerParams(needs_layout_passes=False))


_sc_agg = _make_sc_agg()
_sc_deg = _make_sc_deg()


_R = 1000


def _tc_self(h_self, W_self, b):
    def body(x_ref, ws_ref, b_ref, o_ref):
        o_ref[...] = jnp.dot(x_ref[...], ws_ref[...],
                             preferred_element_type=jnp.float32) + b_ref[...]

    return pl.pallas_call(
        body,
        grid=(N // _R,),
        in_specs=[
            pl.BlockSpec((_R, D), lambda i: (i, 0)),
            pl.BlockSpec((D, D), lambda i: (0, 0)),
            pl.BlockSpec((1, D), lambda i: (0, 0)),
        ],
        out_specs=pl.BlockSpec((_R, D), lambda i: (i, 0)),
        out_shape=jax.ShapeDtypeStruct((N, D), jnp.float32),
    )(h_self, W_self, b.reshape(1, D))


def _tc_combine(xs, agg_parts, deg_parts, W_neigh, relu):
    def body(xs_ref, agg_ref, deg_ref, wn_ref, o_ref):
        agg = agg_ref[0] + agg_ref[1]
        deg = jnp.maximum(deg_ref[0, :, :1] + deg_ref[1, :, :1], 1.0)
        hn = agg / deg
        out = xs_ref[...] + jnp.dot(hn, wn_ref[...],
                                    preferred_element_type=jnp.float32)
        if relu:
            out = jnp.maximum(out, 0.0)
        o_ref[...] = out

    return pl.pallas_call(
        body,
        grid=(N // _R,),
        in_specs=[
            pl.BlockSpec((_R, D), lambda i: (i, 0)),
            pl.BlockSpec((NC, _R, D), lambda i: (0, i, 0)),
            pl.BlockSpec((NC, _R, 1), lambda i: (0, i, 0)),
            pl.BlockSpec((D, D), lambda i: (0, 0)),
        ],
        out_specs=pl.BlockSpec((_R, D), lambda i: (i, 0)),
        out_shape=jax.ShapeDtypeStruct((N, D), jnp.float32),
    )(xs, agg_parts, deg_parts, W_neigh)


def kernel(inputs, edge_index, W_self1, W_neigh1, b1, W_self2, W_neigh2, b2):
    ei = edge_index.astype(jnp.int32)
    src3 = ei[0].reshape(NW, NCH, K)
    dst3 = ei[1].reshape(NW, NCH, K)
    deg = _sc_deg(ei[1].reshape(NW, E_T)).reshape(NC, NPAD)[:, :N, None]
    xs1 = _tc_self(inputs, W_self1, b1)
    agg1 = _sc_agg(inputs, src3, dst3).reshape(NC, N, D)
    h1 = _tc_combine(xs1, agg1, deg, W_neigh1, relu=True)
    xs2 = _tc_self(h1, W_self2, b2)
    agg2 = _sc_agg(h1, src3, dst3).reshape(NC, N, D)
    h2 = _tc_combine(xs2, agg2, deg, W_neigh2, relu=False)
    return h2
